# Initial kernel scaffold; baseline (speedup 1.0000x reference)
import jax
import jax.numpy as jnp
from jax.experimental import pallas as pl


def kernel(x, Wm, bm, Ws, Wq1, bq1, Wq2, bq2):
    raise NotImplementedError("write your pallas kernel here")


# speedup vs baseline: 1.7816x; 1.7816x over previous
import functools

import jax
import jax.numpy as jnp
from jax.experimental import pallas as pl


_D = 256
_L = 7
_MAXP = 32
_BB = 32
_NEG = -1e30


def _rank(s):
    p_dim, bb = s.shape
    sq = s[None, :, :]
    sp = s[:, None, :]
    qi = jax.lax.broadcasted_iota(jnp.int32, (p_dim, p_dim, bb), 1)
    pi = jax.lax.broadcasted_iota(jnp.int32, (p_dim, p_dim, bb), 0)
    better = (sq > sp) | ((sq == sp) & (qi < pi))
    return jnp.sum(better.astype(jnp.float32), axis=1)


def _body(x_ref, wcat_ref, bias_ref, wq1_ref, bq1_ref, wq2_ref, bq2_ref,
          out_ref):
    f32 = jnp.float32
    bf16 = jnp.bfloat16

    def dot16(a, b):
        return jnp.dot(a.astype(bf16), b.astype(bf16),
                       preferred_element_type=f32)

    def score(tokens, j, paths):
        h = dot16(tokens, wq1_ref[j])
        h = jnp.maximum(h + bq1_ref[j], 0.0)
        h3 = h.reshape(paths, _BB, 32).astype(bf16).astype(f32)
        w2 = wq2_ref[j][None].astype(bf16).astype(f32)
        s = jnp.sum(h3 * w2, axis=2)
        return s + bq2_ref[j]

    x = x_ref[...]
    mask64 = None
    for i in range(_L):
        y = dot16(x, wcat_ref[i])
        y = y + bias_ref[i]
        x = jnp.concatenate([y[:, :_D], y[:, _D:]], axis=0)
        if i == _L - 2:
            s5 = score(x, 0, 64)
            mask64 = _rank(s5) < float(_MAXP)
        if i < _L - 1:
            x = jnp.tanh(x)

    s6 = score(x, 1, 128)
    live = jnp.concatenate([mask64, mask64], axis=0)
    s6 = jnp.where(live, s6, _NEG)
    keep = _rank(s6) < float(_MAXP)
    s6 = jnp.where(keep, s6, _NEG)
    m = jnp.max(s6, axis=0, keepdims=True)
    e = jnp.exp(s6 - m)
    w = e / jnp.sum(e, axis=0, keepdims=True)

    wt = w.T
    acc = jnp.zeros((_BB, _D), f32)
    for p in range(128):
        acc = acc + x[p * _BB:(p + 1) * _BB, :] * wt[:, p:p + 1]
    out_ref[...] = acc


@jax.jit
def kernel(x, Wm, bm, Ws, Wq1, bq1, Wq2, bq2):
    batch, d = x.shape
    num_layers = Wm.shape[0]

    wcat = jnp.concatenate(
        [jnp.swapaxes(Wm, 1, 2), jnp.swapaxes(Ws, 1, 2)], axis=2)
    bias = jnp.concatenate([bm, jnp.zeros_like(bm)], axis=1)[:, None, :]
    wq1t = jnp.swapaxes(Wq1[num_layers - 2:], 1, 2)
    bq1s = bq1[num_layers - 2:][:, None, :]
    wq2s = Wq2[num_layers - 2:, 0, :][:, None, :]
    bq2s = bq2[num_layers - 2:][:, :, None]

    grid = (batch // _BB,)
    full = lambda *shape: pl.BlockSpec(shape, lambda i: (0,) * len(shape))
    return pl.pallas_call(
        _body,
        grid=grid,
        in_specs=[
            pl.BlockSpec((_BB, d), lambda i: (i, 0)),
            full(num_layers, d, 2 * d),
            full(num_layers, 1, 2 * d),
            full(2, d, 32),
            full(2, 1, 32),
            full(2, 1, 32),
            full(2, 1, 1),
        ],
        out_specs=pl.BlockSpec((_BB, d), lambda i: (i, 0)),
        out_shape=jax.ShapeDtypeStruct((batch, d), jnp.float32),
    )(x, wcat, bias, wq1t, bq1s, wq2s, bq2s)
